# baseline (speedup 1.0000x reference)
import jax
import jax.numpy as jnp
from jax import lax
from jax.experimental import pallas as pl
from jax.experimental.pallas import tpu as pltpu
from jax.experimental.pallas import tpu_sc as plsc

f32 = jnp.float32
i32 = jnp.int32

E = 320000
N = 10000
D = 128
H = 16
DH = 8
ESD = 32
FH = 64

NS = 2
ES = E // NS
BE = 3200
CH = 40
NW = 32
EPW = ES // NW
RPW = EPW // CH


def _ln(x, g, b):
    mu = jnp.mean(x, axis=-1, keepdims=True)
    var = jnp.mean((x - mu) ** 2, axis=-1, keepdims=True)
    return (x - mu) * jax.lax.rsqrt(var + 1e-5) * g + b


def _dot(x, w_r):
    return jnp.dot(x, w_r[...], preferred_element_type=f32)


def _spread16():
    rr = lax.broadcasted_iota(i32, (H, D), 0)
    dd = lax.broadcasted_iota(i32, (H, D), 1) // DH
    return (rr == dd).astype(f32)


def _tc1_body(msg_r, ea_r, es_r, W0_r, b0_r, g0_r, bt0_r, W1_r, b1_r, g1_r,
              bt1_r, W2_r, Wa_r, ba_r, Wl_r, bl_r, wd2_r, Wv_r, bv_r, adf_r,
              ex_o, attn_o):
    x = _dot(es_r[...], W0_r) + b0_r[...]
    x = _ln(x, g0_r[...], bt0_r[...])
    x = x * jax.nn.sigmoid(x)
    x = _dot(x, W1_r) + b1_r[...]
    x = _ln(x, g1_r[...], bt1_r[...])
    x = x * jax.nn.sigmoid(x)
    w = _dot(x, W2_r)
    m = msg_r[...] * ea_r[...] * w
    a = _dot(m, Wa_r) + ba_r[...]
    a = 0.6 * a + 0.4 * a * (2.0 * jax.nn.sigmoid(a) - 1.0)
    hh = lax.broadcasted_iota(i32, (D, H), 0) // DH
    cc = lax.broadcasted_iota(i32, (D, H), 1)
    gather16 = (hh == cc).astype(f32)
    lb16 = jnp.dot(a * adf_r[...], gather16, preferred_element_type=f32)
    ex16 = jnp.exp(lb16)
    exv = jnp.dot(ex16, _spread16(), preferred_element_type=f32)
    ex_o[...] = exv
    v = _dot(m, Wl_r) + bl_r[...]
    v = v * jax.nn.sigmoid(v)
    v = v * ea_r[...] * wd2_r[...]
    attn_o[...] = exv * (_dot(v, Wv_r) + bv_r[...])


def _tc3_body(*refs):
    num_rs = refs[:2]
    den_rs = refs[2:4]
    Wp_r, bp_r, sc_r, out_r = refs[4:]
    num = sum(r[0] + r[1] for r in num_rs)
    den = sum(r[0] + r[1] for r in den_rs)
    node = num * (sc_r[0, 0] / (den + 1e-16))
    out_r[...] = jnp.dot(node, Wp_r[...], preferred_element_type=f32) + bp_r[...]


def _sc_mesh():
    return plsc.VectorSubcoreMesh(core_axis_name="c", subcore_axis_name="s")


def _wid():
    return lax.axis_index("s") * 2 + lax.axis_index("c")


def _ring(nb, bufs, sems_a, sems_b, mk_a, mk_b):
    for b in range(nb):
        mk_a(b, bufs[b], sems_a[b])

    def group(g, carry):
        for b in range(nb):
            r = g * nb + b
            mk_a(r, bufs[b], sems_a[b], wait=True)
            mk_b(r, bufs[b], sems_b[b])
            mk_b(r, bufs[b], sems_b[b], wait=True)

            @pl.when(r + nb < RPW)
            def _():
                mk_a(r + nb, bufs[b], sems_a[b])

        return carry

    lax.fori_loop(0, RPW // nb, group, 0)
    for r in range((RPW // nb) * nb, RPW):
        b = r % nb
        mk_a(r, bufs[b], sems_a[b], wait=True)
        mk_b(r, bufs[b], sems_b[b])
        mk_b(r, bufs[b], sems_b[b], wait=True)


def _copy(src, dst, sem, wait):
    if wait:
        pltpu.make_async_copy(src, dst, sem).wait()
    else:
        pltpu.async_copy(src, dst, sem)


_STRIPE = 624
_TAIL = N - 15 * _STRIPE

NBA = 4


def _stripe_chunks(start, rows):
    off = 0
    while off < rows:
        sz = min(CH, rows - off)
        yield pl.multiple_of(start + off, 8), sz
        off += sz


def _accum_body(dst_r, src_r, zer_r, out_o, tab_sh, *rest):
    bufs = rest[:NBA]
    idxb = rest[NBA:2 * NBA]
    sems_a = rest[2 * NBA:3 * NBA]
    sems_b = rest[3 * NBA:4 * NBA]
    c = lax.axis_index("c")
    s = lax.axis_index("s")
    wid = _wid()
    st0 = pl.multiple_of(s * _STRIPE, 8)

    pltpu.sync_copy(zer_r, bufs[0])

    @pl.when(s < 15)
    def _():
        for off, sz in _stripe_chunks(st0, _STRIPE):
            pltpu.sync_copy(bufs[0].at[pl.ds(0, sz)], tab_sh.at[pl.ds(off, sz)])

    @pl.when(s == 15)
    def _():
        for off, sz in _stripe_chunks(15 * _STRIPE, _TAIL):
            pltpu.sync_copy(bufs[0].at[pl.ds(0, sz)], tab_sh.at[pl.ds(off, sz)])

    plsc.subcore_barrier()

    slot = {id(b): k for k, b in enumerate(bufs)}

    def mk_a(r, buf, sem, wait=False):
        e0 = pl.multiple_of((wid * RPW + r) * CH, 8)
        _copy(src_r.at[pl.ds(e0, CH)], buf, sem, wait)
        _copy(dst_r.at[wid, r], idxb[slot[id(buf)]], sem, wait)

    def mk_b(r, buf, sem, wait=False):
        ib = idxb[slot[id(buf)]].at[0]
        if wait:
            pltpu.make_async_copy(buf, tab_sh.at[ib], sem).wait()
        else:
            pltpu.async_copy(buf, tab_sh.at[ib], sem, add=True)

    _ring(NBA, bufs, sems_a, sems_b, mk_a, mk_b)
    plsc.subcore_barrier()

    @pl.when(s < 15)
    def _():
        for off, sz in _stripe_chunks(st0, _STRIPE):
            pltpu.sync_copy(tab_sh.at[pl.ds(off, sz)], bufs[0].at[pl.ds(0, sz)])
            pltpu.sync_copy(bufs[0].at[pl.ds(0, sz)], out_o.at[c].at[pl.ds(off, sz)])

    @pl.when(s == 15)
    def _():
        for off, sz in _stripe_chunks(15 * _STRIPE, _TAIL):
            pltpu.sync_copy(tab_sh.at[pl.ds(off, sz)], bufs[0].at[pl.ds(0, sz)])
            pltpu.sync_copy(bufs[0].at[pl.ds(0, sz)], out_o.at[c].at[pl.ds(off, sz)])


def _full(shape):
    return pl.BlockSpec(shape, lambda i: (0, 0))


def kernel(message, edge_dst, edge_attr, edge_scalars, n_nodes_dst,
           W0, b0, g0, bt0, W1, b1, g1, bt1, W2,
           W_alpha, b_alpha, W_lin, b_lin, w_dtp2, W_val, b_val,
           alpha_dot, W_proj, b_proj):
    adf = alpha_dot.reshape(1, D)
    nblk = ES // BE

    accum = pl.kernel(
        _accum_body,
        out_type=jax.ShapeDtypeStruct((2, N, D), f32),
        mesh=_sc_mesh(),
        scratch_types=[pltpu.VMEM_SHARED((N, D), f32)]
                      + [pltpu.VMEM((CH, D), f32)] * NBA
                      + [pltpu.VMEM((1, CH), i32)] * NBA
                      + [pltpu.SemaphoreType.DMA] * (2 * NBA),
    )
    zeros_stripe = jnp.zeros((CH, D), f32)

    nums, dens = [], []
    for k in range(NS):
        eb = lambda w, k=k: pl.BlockSpec((BE, w), lambda i: (i + k * nblk, 0))
        ex, attn = pl.pallas_call(
            _tc1_body,
            grid=(nblk,),
            in_specs=[eb(D), eb(1), eb(ESD),
                      _full((ESD, FH)), _full((1, FH)), _full((1, FH)),
                      _full((1, FH)),
                      _full((FH, FH)), _full((1, FH)), _full((1, FH)),
                      _full((1, FH)),
                      _full((FH, D)),
                      _full((D, D)), _full((1, D)),
                      _full((D, D)), _full((1, D)),
                      _full((1, D)),
                      _full((D, D)), _full((1, D)),
                      _full((1, D))],
            out_specs=[pl.BlockSpec((BE, D), lambda i: (i, 0)),
                       pl.BlockSpec((BE, D), lambda i: (i, 0))],
            out_shape=[jax.ShapeDtypeStruct((ES, D), f32),
                       jax.ShapeDtypeStruct((ES, D), f32)],
        )(message, edge_attr, edge_scalars,
          W0, b0.reshape(1, FH), g0.reshape(1, FH), bt0.reshape(1, FH),
          W1, b1.reshape(1, FH), g1.reshape(1, FH), bt1.reshape(1, FH),
          W2, W_alpha, b_alpha.reshape(1, D), W_lin, b_lin.reshape(1, D),
          w_dtp2.reshape(1, D), W_val, b_val.reshape(1, D), adf)
        dst4 = lax.slice_in_dim(edge_dst, k * ES, (k + 1) * ES).reshape(
            NW, RPW, 1, CH)
        nums.append(accum(dst4, attn, zeros_stripe))
        dens.append(accum(dst4, ex, zeros_stripe))

    scale = jnp.asarray(n_nodes_dst, f32).reshape(1, 1) / float(N)
    p2 = lambda: pl.BlockSpec((2, N, D), lambda: (0, 0, 0))
    out = pl.pallas_call(
        _tc3_body,
        in_specs=[p2() for _ in range(2 * NS)]
                 + [pl.BlockSpec((D, D), lambda: (0, 0)),
                    pl.BlockSpec((1, D), lambda: (0, 0)),
                    pl.BlockSpec((1, 1), lambda: (0, 0))],
        out_specs=pl.BlockSpec((N, D), lambda: (0, 0)),
        out_shape=jax.ShapeDtypeStruct((N, D), f32),
    )(*nums, *dens, W_proj, b_proj.reshape(1, D), scale)
    return out


# speedup vs baseline: 1.0186x; 1.0186x over previous
import jax
import jax.numpy as jnp
from jax import lax
from jax.experimental import pallas as pl
from jax.experimental.pallas import tpu as pltpu
from jax.experimental.pallas import tpu_sc as plsc

f32 = jnp.float32
i32 = jnp.int32

E = 320000
N = 10000
D = 128
H = 16
DH = 8
ESD = 32
FH = 64

NS = 1
ES = E // NS
BE = 3200
CH = 80
NW = 32
EPW = ES // NW
RPW = EPW // CH


def _ln(x, g, b):
    mu = jnp.mean(x, axis=-1, keepdims=True)
    var = jnp.mean((x - mu) ** 2, axis=-1, keepdims=True)
    return (x - mu) * jax.lax.rsqrt(var + 1e-5) * g + b


def _dot(x, w_r):
    return jnp.dot(x, w_r[...], preferred_element_type=f32)


def _spread16():
    rr = lax.broadcasted_iota(i32, (H, D), 0)
    dd = lax.broadcasted_iota(i32, (H, D), 1) // DH
    return (rr == dd).astype(f32)


def _tc1_body(msg_r, ea_r, es_r, W0_r, b0_r, g0_r, bt0_r, W1_r, b1_r, g1_r,
              bt1_r, W2_r, Wa_r, ba_r, Wl_r, bl_r, wd2_r, Wv_r, bv_r, adf_r,
              ex_o, attn_o):
    x = _dot(es_r[...], W0_r) + b0_r[...]
    x = _ln(x, g0_r[...], bt0_r[...])
    x = x * jax.nn.sigmoid(x)
    x = _dot(x, W1_r) + b1_r[...]
    x = _ln(x, g1_r[...], bt1_r[...])
    x = x * jax.nn.sigmoid(x)
    w = _dot(x, W2_r)
    m = msg_r[...] * ea_r[...] * w
    a = _dot(m, Wa_r) + ba_r[...]
    a = 0.6 * a + 0.4 * a * (2.0 * jax.nn.sigmoid(a) - 1.0)
    hh = lax.broadcasted_iota(i32, (D, H), 0) // DH
    cc = lax.broadcasted_iota(i32, (D, H), 1)
    gather16 = (hh == cc).astype(f32)
    lb16 = jnp.dot(a * adf_r[...], gather16, preferred_element_type=f32)
    ex16 = jnp.exp(lb16)
    exv = jnp.dot(ex16, _spread16(), preferred_element_type=f32)
    ex_o[...] = exv
    v = _dot(m, Wl_r) + bl_r[...]
    v = v * jax.nn.sigmoid(v)
    v = v * ea_r[...] * wd2_r[...]
    attn_o[...] = exv * (_dot(v, Wv_r) + bv_r[...])


def _tc3_body(*refs):
    num_rs = refs[:NS]
    den_rs = refs[NS:2 * NS]
    Wp_r, bp_r, sc_r, out_r = refs[2 * NS:]
    num = sum(r[0] + r[1] for r in num_rs)
    den = sum(r[0] + r[1] for r in den_rs)
    node = num * (sc_r[0, 0] / (den + 1e-16))
    out_r[...] = jnp.dot(node, Wp_r[...], preferred_element_type=f32) + bp_r[...]


def _sc_mesh():
    return plsc.VectorSubcoreMesh(core_axis_name="c", subcore_axis_name="s")


def _wid():
    return lax.axis_index("s") * 2 + lax.axis_index("c")


def _ring(nb, bufs, sems_a, sems_b, mk_a, mk_b):
    for b in range(nb):
        mk_a(b, bufs[b], sems_a[b])

    def group(g, carry):
        for b in range(nb):
            r = g * nb + b
            mk_a(r, bufs[b], sems_a[b], wait=True)
            mk_b(r, bufs[b], sems_b[b])
            mk_b(r, bufs[b], sems_b[b], wait=True)

            @pl.when(r + nb < RPW)
            def _():
                mk_a(r + nb, bufs[b], sems_a[b])

        return carry

    lax.fori_loop(0, RPW // nb, group, 0)
    for r in range((RPW // nb) * nb, RPW):
        b = r % nb
        mk_a(r, bufs[b], sems_a[b], wait=True)
        mk_b(r, bufs[b], sems_b[b])
        mk_b(r, bufs[b], sems_b[b], wait=True)


def _copy(src, dst, sem, wait):
    if wait:
        pltpu.make_async_copy(src, dst, sem).wait()
    else:
        pltpu.async_copy(src, dst, sem)


_STRIPE = 624
_TAIL = N - 15 * _STRIPE

NBA = 4


def _stripe_chunks(start, rows):
    off = 0
    while off < rows:
        sz = min(CH, rows - off)
        yield pl.multiple_of(start + off, 8), sz
        off += sz


def _accum_body(dst_r, src_r, zer_r, out_o, tab_sh, *rest):
    bufs = rest[:NBA]
    idxb = rest[NBA:2 * NBA]
    sems_a = rest[2 * NBA:3 * NBA]
    sems_b = rest[3 * NBA:4 * NBA]
    c = lax.axis_index("c")
    s = lax.axis_index("s")
    wid = _wid()
    st0 = pl.multiple_of(s * _STRIPE, 8)

    pltpu.sync_copy(zer_r, bufs[0])

    @pl.when(s < 15)
    def _():
        for off, sz in _stripe_chunks(st0, _STRIPE):
            pltpu.sync_copy(bufs[0].at[pl.ds(0, sz)], tab_sh.at[pl.ds(off, sz)])

    @pl.when(s == 15)
    def _():
        for off, sz in _stripe_chunks(15 * _STRIPE, _TAIL):
            pltpu.sync_copy(bufs[0].at[pl.ds(0, sz)], tab_sh.at[pl.ds(off, sz)])

    plsc.subcore_barrier()

    slot = {id(b): k for k, b in enumerate(bufs)}

    def mk_a(r, buf, sem, wait=False):
        e0 = pl.multiple_of((wid * RPW + r) * CH, 8)
        _copy(src_r.at[pl.ds(e0, CH)], buf, sem, wait)
        _copy(dst_r.at[wid, r], idxb[slot[id(buf)]], sem, wait)

    def mk_b(r, buf, sem, wait=False):
        ib = idxb[slot[id(buf)]].at[0]
        if wait:
            pltpu.make_async_copy(buf, tab_sh.at[ib], sem).wait()
        else:
            pltpu.async_copy(buf, tab_sh.at[ib], sem, add=True)

    _ring(NBA, bufs, sems_a, sems_b, mk_a, mk_b)
    plsc.subcore_barrier()

    @pl.when(s < 15)
    def _():
        for off, sz in _stripe_chunks(st0, _STRIPE):
            pltpu.sync_copy(tab_sh.at[pl.ds(off, sz)], bufs[0].at[pl.ds(0, sz)])
            pltpu.sync_copy(bufs[0].at[pl.ds(0, sz)], out_o.at[c].at[pl.ds(off, sz)])

    @pl.when(s == 15)
    def _():
        for off, sz in _stripe_chunks(15 * _STRIPE, _TAIL):
            pltpu.sync_copy(tab_sh.at[pl.ds(off, sz)], bufs[0].at[pl.ds(0, sz)])
            pltpu.sync_copy(bufs[0].at[pl.ds(0, sz)], out_o.at[c].at[pl.ds(off, sz)])


def _full(shape):
    return pl.BlockSpec(shape, lambda i: (0, 0))


def kernel(message, edge_dst, edge_attr, edge_scalars, n_nodes_dst,
           W0, b0, g0, bt0, W1, b1, g1, bt1, W2,
           W_alpha, b_alpha, W_lin, b_lin, w_dtp2, W_val, b_val,
           alpha_dot, W_proj, b_proj):
    adf = alpha_dot.reshape(1, D)
    nblk = ES // BE

    accum = pl.kernel(
        _accum_body,
        out_type=jax.ShapeDtypeStruct((2, N, D), f32),
        mesh=_sc_mesh(),
        scratch_types=[pltpu.VMEM_SHARED((N, D), f32)]
                      + [pltpu.VMEM((CH, D), f32)] * NBA
                      + [pltpu.VMEM((1, CH), i32)] * NBA
                      + [pltpu.SemaphoreType.DMA] * (2 * NBA),
    )
    zeros_stripe = jnp.zeros((CH, D), f32)

    nums, dens = [], []
    for k in range(NS):
        eb = lambda w, k=k: pl.BlockSpec((BE, w), lambda i: (i + k * nblk, 0))
        ex, attn = pl.pallas_call(
            _tc1_body,
            grid=(nblk,),
            in_specs=[eb(D), eb(1), eb(ESD),
                      _full((ESD, FH)), _full((1, FH)), _full((1, FH)),
                      _full((1, FH)),
                      _full((FH, FH)), _full((1, FH)), _full((1, FH)),
                      _full((1, FH)),
                      _full((FH, D)),
                      _full((D, D)), _full((1, D)),
                      _full((D, D)), _full((1, D)),
                      _full((1, D)),
                      _full((D, D)), _full((1, D)),
                      _full((1, D))],
            out_specs=[pl.BlockSpec((BE, D), lambda i: (i, 0)),
                       pl.BlockSpec((BE, D), lambda i: (i, 0))],
            out_shape=[jax.ShapeDtypeStruct((ES, D), f32),
                       jax.ShapeDtypeStruct((ES, D), f32)],
        )(message, edge_attr, edge_scalars,
          W0, b0.reshape(1, FH), g0.reshape(1, FH), bt0.reshape(1, FH),
          W1, b1.reshape(1, FH), g1.reshape(1, FH), bt1.reshape(1, FH),
          W2, W_alpha, b_alpha.reshape(1, D), W_lin, b_lin.reshape(1, D),
          w_dtp2.reshape(1, D), W_val, b_val.reshape(1, D), adf)
        dst4 = lax.slice_in_dim(edge_dst, k * ES, (k + 1) * ES).reshape(
            NW, RPW, 1, CH)
        nums.append(accum(dst4, attn, zeros_stripe))
        dens.append(accum(dst4, ex, zeros_stripe))

    scale = jnp.asarray(n_nodes_dst, f32).reshape(1, 1) / float(N)
    p2 = lambda: pl.BlockSpec((2, N, D), lambda: (0, 0, 0))
    out = pl.pallas_call(
        _tc3_body,
        in_specs=[p2() for _ in range(2 * NS)]
                 + [pl.BlockSpec((D, D), lambda: (0, 0)),
                    pl.BlockSpec((1, D), lambda: (0, 0)),
                    pl.BlockSpec((1, 1), lambda: (0, 0))],
        out_specs=pl.BlockSpec((N, D), lambda: (0, 0)),
        out_shape=jax.ShapeDtypeStruct((N, D), f32),
    )(*nums, *dens, W_proj, b_proj.reshape(1, D), scale)
    return out


# speedup vs baseline: 1.1793x; 1.1578x over previous
import jax
import jax.numpy as jnp
from jax import lax
from jax.experimental import pallas as pl
from jax.experimental.pallas import tpu as pltpu
from jax.experimental.pallas import tpu_sc as plsc

f32 = jnp.float32
i32 = jnp.int32

E = 320000
N = 10000
D = 128
H = 16
DH = 8
ESD = 32
FH = 64

NS = 1
ES = E // NS
BE = 3200
CH = 80
NW = 32
EPW = ES // NW
RPW = EPW // CH


def _ln(x, g, b):
    mu = jnp.mean(x, axis=-1, keepdims=True)
    var = jnp.mean((x - mu) ** 2, axis=-1, keepdims=True)
    return (x - mu) * jax.lax.rsqrt(var + 1e-5) * g + b


def _dot(x, w_r):
    return jnp.dot(x, w_r[...], preferred_element_type=f32)


def _spread16():
    rr = lax.broadcasted_iota(i32, (H, D), 0)
    dd = lax.broadcasted_iota(i32, (H, D), 1) // DH
    return (rr == dd).astype(f32)


def _groupavg():
    rr = lax.broadcasted_iota(i32, (D, D), 0) // FH
    cc = lax.broadcasted_iota(i32, (D, D), 1) // FH
    return (rr == cc).astype(f32) * (1.0 / FH)


def _lnp(x, g, b):
    gm = _groupavg()
    mu = jnp.dot(x, gm, preferred_element_type=f32)
    xc = x - mu
    var = jnp.dot(xc * xc, gm, preferred_element_type=f32)
    return xc * jax.lax.rsqrt(var + 1e-5) * g + b


def _dup(v):
    return jnp.concatenate([v, v], axis=1)


def _tc1_body(msg_r, ea_r, es_r, W0_r, b0_r, g0_r, bt0_r, W1_r, b1_r, g1_r,
              bt1_r, W2_r, Wa_r, ba_r, Wl_r, bl_r, wd2_r, Wv_r, bv_r, adf_r,
              ex_o, attn_o):
    be2 = BE // 2
    es = es_r[...]
    xcat = jnp.concatenate([es[:be2], es[be2:]], axis=1)
    W0 = W0_r[...]
    z0 = jnp.zeros((ESD, FH), f32)
    W0b = jnp.concatenate([jnp.concatenate([W0, z0], 0),
                           jnp.concatenate([z0, W0], 0)], 1)
    x = jnp.dot(xcat, W0b, preferred_element_type=f32) + _dup(b0_r[...])
    x = _lnp(x, _dup(g0_r[...]), _dup(bt0_r[...]))
    x = x * jax.nn.sigmoid(x)
    W1 = W1_r[...]
    z1 = jnp.zeros((FH, FH), f32)
    W1b = jnp.concatenate([jnp.concatenate([W1, z1], 0),
                           jnp.concatenate([z1, W1], 0)], 1)
    x = jnp.dot(x, W1b, preferred_element_type=f32) + _dup(b1_r[...])
    x = _lnp(x, _dup(g1_r[...]), _dup(bt1_r[...]))
    x = x * jax.nn.sigmoid(x)
    W2 = W2_r[...]
    z2 = jnp.zeros((FH, D), f32)
    wA = jnp.dot(x, jnp.concatenate([W2, z2], 0), preferred_element_type=f32)
    wB = jnp.dot(x, jnp.concatenate([z2, W2], 0), preferred_element_type=f32)
    w = jnp.concatenate([wA, wB], axis=0)
    m = msg_r[...] * ea_r[...] * w
    a = _dot(m, Wa_r) + ba_r[...]
    a = 0.6 * a + 0.4 * a * (2.0 * jax.nn.sigmoid(a) - 1.0)
    hh = lax.broadcasted_iota(i32, (D, H), 0) // DH
    cc = lax.broadcasted_iota(i32, (D, H), 1)
    gather16 = (hh == cc).astype(f32)
    lb16 = jnp.dot(a * adf_r[...], gather16, preferred_element_type=f32)
    ex16 = jnp.exp(lb16)
    exv = jnp.dot(ex16, _spread16(), preferred_element_type=f32)
    ex_o[...] = exv
    v = _dot(m, Wl_r) + bl_r[...]
    v = v * jax.nn.sigmoid(v)
    v = v * ea_r[...] * wd2_r[...]
    attn_o[...] = exv * (_dot(v, Wv_r) + bv_r[...])


def _tc3_body(*refs):
    num_rs = refs[:NS]
    den_rs = refs[NS:2 * NS]
    Wp_r, bp_r, sc_r, out_r = refs[2 * NS:]
    num = sum(r[0] + r[1] for r in num_rs)
    den = sum(r[0] + r[1] for r in den_rs)
    node = num * (sc_r[0, 0] / (den + 1e-16))
    out_r[...] = jnp.dot(node, Wp_r[...], preferred_element_type=f32) + bp_r[...]


def _sc_mesh():
    return plsc.VectorSubcoreMesh(core_axis_name="c", subcore_axis_name="s")


def _wid():
    return lax.axis_index("s") * 2 + lax.axis_index("c")


def _ring(nb, bufs, sems_a, sems_b, mk_a, mk_b):
    for b in range(nb):
        mk_a(b, bufs[b], sems_a[b])

    def group(g, carry):
        for b in range(nb):
            r = g * nb + b
            mk_a(r, bufs[b], sems_a[b], wait=True)
            mk_b(r, bufs[b], sems_b[b])
            mk_b(r, bufs[b], sems_b[b], wait=True)

            @pl.when(r + nb < RPW)
            def _():
                mk_a(r + nb, bufs[b], sems_a[b])

        return carry

    lax.fori_loop(0, RPW // nb, group, 0)
    for r in range((RPW // nb) * nb, RPW):
        b = r % nb
        mk_a(r, bufs[b], sems_a[b], wait=True)
        mk_b(r, bufs[b], sems_b[b])
        mk_b(r, bufs[b], sems_b[b], wait=True)


def _copy(src, dst, sem, wait):
    if wait:
        pltpu.make_async_copy(src, dst, sem).wait()
    else:
        pltpu.async_copy(src, dst, sem)


_STRIPE = 624
_TAIL = N - 15 * _STRIPE

NBA = 4


def _stripe_chunks(start, rows):
    off = 0
    while off < rows:
        sz = min(CH, rows - off)
        yield pl.multiple_of(start + off, 8), sz
        off += sz


def _accum_body(dst_r, src_r, zer_r, out_o, tab_sh, *rest):
    bufs = rest[:NBA]
    idxb = rest[NBA:2 * NBA]
    sems_a = rest[2 * NBA:3 * NBA]
    sems_b = rest[3 * NBA:4 * NBA]
    c = lax.axis_index("c")
    s = lax.axis_index("s")
    wid = _wid()
    st0 = pl.multiple_of(s * _STRIPE, 8)

    pltpu.sync_copy(zer_r, bufs[0])

    @pl.when(s < 15)
    def _():
        for off, sz in _stripe_chunks(st0, _STRIPE):
            pltpu.sync_copy(bufs[0].at[pl.ds(0, sz)], tab_sh.at[pl.ds(off, sz)])

    @pl.when(s == 15)
    def _():
        for off, sz in _stripe_chunks(15 * _STRIPE, _TAIL):
            pltpu.sync_copy(bufs[0].at[pl.ds(0, sz)], tab_sh.at[pl.ds(off, sz)])

    plsc.subcore_barrier()

    slot = {id(b): k for k, b in enumerate(bufs)}

    def mk_a(r, buf, sem, wait=False):
        e0 = pl.multiple_of((wid * RPW + r) * CH, 8)
        _copy(src_r.at[pl.ds(e0, CH)], buf, sem, wait)
        _copy(dst_r.at[wid, r], idxb[slot[id(buf)]], sem, wait)

    def mk_b(r, buf, sem, wait=False):
        ib = idxb[slot[id(buf)]].at[0]
        if wait:
            pltpu.make_async_copy(buf, tab_sh.at[ib], sem).wait()
        else:
            pltpu.async_copy(buf, tab_sh.at[ib], sem, add=True)

    _ring(NBA, bufs, sems_a, sems_b, mk_a, mk_b)
    plsc.subcore_barrier()

    @pl.when(s < 15)
    def _():
        for off, sz in _stripe_chunks(st0, _STRIPE):
            pltpu.sync_copy(tab_sh.at[pl.ds(off, sz)], bufs[0].at[pl.ds(0, sz)])
            pltpu.sync_copy(bufs[0].at[pl.ds(0, sz)], out_o.at[c].at[pl.ds(off, sz)])

    @pl.when(s == 15)
    def _():
        for off, sz in _stripe_chunks(15 * _STRIPE, _TAIL):
            pltpu.sync_copy(tab_sh.at[pl.ds(off, sz)], bufs[0].at[pl.ds(0, sz)])
            pltpu.sync_copy(bufs[0].at[pl.ds(0, sz)], out_o.at[c].at[pl.ds(off, sz)])


def _full(shape):
    return pl.BlockSpec(shape, lambda i: (0, 0))


def kernel(message, edge_dst, edge_attr, edge_scalars, n_nodes_dst,
           W0, b0, g0, bt0, W1, b1, g1, bt1, W2,
           W_alpha, b_alpha, W_lin, b_lin, w_dtp2, W_val, b_val,
           alpha_dot, W_proj, b_proj):
    adf = alpha_dot.reshape(1, D)
    nblk = ES // BE

    accum = pl.kernel(
        _accum_body,
        out_type=jax.ShapeDtypeStruct((2, N, D), f32),
        mesh=_sc_mesh(),
        scratch_types=[pltpu.VMEM_SHARED((N, D), f32)]
                      + [pltpu.VMEM((CH, D), f32)] * NBA
                      + [pltpu.VMEM((1, CH), i32)] * NBA
                      + [pltpu.SemaphoreType.DMA] * (2 * NBA),
    )
    zeros_stripe = jnp.zeros((CH, D), f32)

    nums, dens = [], []
    for k in range(NS):
        eb = lambda w, k=k: pl.BlockSpec((BE, w), lambda i: (i + k * nblk, 0))
        ex, attn = pl.pallas_call(
            _tc1_body,
            grid=(nblk,),
            in_specs=[eb(D), eb(1), eb(ESD),
                      _full((ESD, FH)), _full((1, FH)), _full((1, FH)),
                      _full((1, FH)),
                      _full((FH, FH)), _full((1, FH)), _full((1, FH)),
                      _full((1, FH)),
                      _full((FH, D)),
                      _full((D, D)), _full((1, D)),
                      _full((D, D)), _full((1, D)),
                      _full((1, D)),
                      _full((D, D)), _full((1, D)),
                      _full((1, D))],
            out_specs=[pl.BlockSpec((BE, D), lambda i: (i, 0)),
                       pl.BlockSpec((BE, D), lambda i: (i, 0))],
            out_shape=[jax.ShapeDtypeStruct((ES, D), f32),
                       jax.ShapeDtypeStruct((ES, D), f32)],
        )(message, edge_attr, edge_scalars,
          W0, b0.reshape(1, FH), g0.reshape(1, FH), bt0.reshape(1, FH),
          W1, b1.reshape(1, FH), g1.reshape(1, FH), bt1.reshape(1, FH),
          W2, W_alpha, b_alpha.reshape(1, D), W_lin, b_lin.reshape(1, D),
          w_dtp2.reshape(1, D), W_val, b_val.reshape(1, D), adf)
        dst4 = lax.slice_in_dim(edge_dst, k * ES, (k + 1) * ES).reshape(
            NW, RPW, 1, CH)
        nums.append(accum(dst4, attn, zeros_stripe))
        dens.append(accum(dst4, ex, zeros_stripe))

    scale = jnp.asarray(n_nodes_dst, f32).reshape(1, 1) / float(N)
    p2 = lambda: pl.BlockSpec((2, N, D), lambda: (0, 0, 0))
    out = pl.pallas_call(
        _tc3_body,
        in_specs=[p2() for _ in range(2 * NS)]
                 + [pl.BlockSpec((D, D), lambda: (0, 0)),
                    pl.BlockSpec((1, D), lambda: (0, 0)),
                    pl.BlockSpec((1, 1), lambda: (0, 0))],
        out_specs=pl.BlockSpec((N, D), lambda: (0, 0)),
        out_shape=jax.ShapeDtypeStruct((N, D), f32),
    )(*nums, *dens, W_proj, b_proj.reshape(1, D), scale)
    return out
